# baseline (speedup 1.0000x reference)
import functools

import jax
import jax.numpy as jnp
from jax import lax
from jax.experimental import pallas as pl
from jax.experimental.pallas import tpu as pltpu
from jax.experimental.pallas import tpu_sc as plsc

N = 10000
E = 320000
D = 128
H = 64
T = 10
TN = 10
NCLS = 10
EPS = 0.1
SINK_ITERS = 20

NC = 2
NS = 16
NW = NC * NS

CHUNK = 128
NCHUNK = 80
EPW = NCHUNK * CHUNK
E_PAD = EPW * NW
NACC = N + 112
ZR = NACC // NS

def _deg_body(dst_hbm, out_hbm, idx_v, ones_v, zer_v, acc_sh):
    cid = lax.axis_index("c")
    sid = lax.axis_index("s")
    wid = cid * NS + sid
    for k in range(CHUNK // 16):
        ones_v[pl.ds(k * 16, 16)] = jnp.ones((16,), jnp.float32)

    for k in range(ZR // 16):
        zer_v[pl.ds(k * 16, 16)] = jnp.zeros((16,), jnp.float32)
    if ZR % 16:
        zer_v[pl.ds(ZR - 16, 16)] = jnp.zeros((16,), jnp.float32)

    pltpu.sync_copy(zer_v, acc_sh.at[pl.ds(sid * ZR, ZR)])
    plsc.subcore_barrier()
    pltpu.sync_copy(dst_hbm.at[wid], idx_v)

    def body(j, carry):
        pltpu.sync_copy(ones_v, acc_sh.at[idx_v.at[j]], add=True)
        return carry

    lax.fori_loop(0, NCHUNK, body, 0)
    plsc.subcore_barrier()
    pltpu.sync_copy(acc_sh.at[pl.ds(sid * ZR, ZR)], zer_v)
    pltpu.sync_copy(zer_v, out_hbm.at[pl.ds(cid * NACC + sid * ZR, ZR)])


NB = 8
LOOK = 4


HPW = H // 2


def _stage_rows(hpb_hbm, hpb_sh, rows_v, base, nrows):
    for q in range(nrows // CHUNK):
        pltpu.sync_copy(hpb_hbm.at[pl.ds(base + q * CHUNK, CHUNK)],
                        rows_v.at[q % NB])
        pltpu.sync_copy(rows_v.at[q % NB],
                        hpb_sh.at[pl.ds(base + q * CHUNK, CHUNK)])
    rem = nrows % CHUNK
    if rem:
        full = (nrows // CHUNK) * CHUNK
        pltpu.sync_copy(hpb_hbm.at[pl.ds(base + full, rem)],
                        rows_v.at[0].at[pl.ds(0, rem)])
        pltpu.sync_copy(rows_v.at[0].at[pl.ds(0, rem)],
                        hpb_sh.at[pl.ds(base + full, rem)])


def _agg_body(hpb_hbm, src_hbm, dst_hbm, out_hbm,
              si_v, di_v, rows_v, hpb_sh, acc_sh, *sems):
    gsems = sems[:NB]
    ssems = sems[NB:]
    cid = lax.axis_index("c")
    sid = lax.axis_index("s")
    wid = cid * NS + sid

    @pl.when(sid < NS - 1)
    def _():
        _stage_rows(hpb_hbm, hpb_sh, rows_v, sid * ZR, ZR)

    @pl.when(sid == NS - 1)
    def _():
        _stage_rows(hpb_hbm, hpb_sh, rows_v, (NS - 1) * ZR, N - (NS - 1) * ZR)

    def zrow(r, carry):
        for k in range(H // 32):
            rows_v[0, r, pl.ds(k * 32, 32)] = jnp.zeros((32,), jnp.bfloat16)
        return carry

    lax.fori_loop(0, CHUNK, zrow, 0)
    for q in range(ZR // CHUNK):
        pltpu.sync_copy(rows_v.at[0],
                        acc_sh.at[pl.ds(sid * ZR + q * CHUNK, CHUNK)])
    rem = ZR % CHUNK
    if rem:
        pltpu.sync_copy(rows_v.at[0].at[pl.ds(0, rem)],
                        acc_sh.at[pl.ds(sid * ZR + (ZR // CHUNK) * CHUNK, rem)])
    plsc.subcore_barrier()
    pltpu.sync_copy(src_hbm.at[wid], si_v)
    pltpu.sync_copy(dst_hbm.at[wid], di_v)

    for j in range(LOOK):
        pltpu.async_copy(hpb_sh.at[si_v.at[j]], rows_v.at[j % NB],
                         gsems[j % NB])

    def outer(t, carry):
        for b in range(NB):
            j = t * NB + b
            pltpu.make_async_copy(hpb_sh.at[si_v.at[j]], rows_v.at[b],
                                  gsems[b]).wait()
            pltpu.async_copy(rows_v.at[b], acc_sh.at[di_v.at[j]], ssems[b],
                             add=True)
            tgt = j + LOOK
            tb = (b + LOOK) % NB

            @pl.when(tgt >= NB)
            def _():
                pltpu.make_async_copy(rows_v.at[tb],
                                      acc_sh.at[di_v.at[tgt - NB]],
                                      ssems[tb]).wait()

            @pl.when(tgt < NCHUNK)
            def _():
                pltpu.async_copy(hpb_sh.at[si_v.at[tgt]], rows_v.at[tb],
                                 gsems[tb])
        return carry

    lax.fori_loop(0, NCHUNK // NB, outer, 0)
    for j in range(NCHUNK - LOOK, NCHUNK):
        b = j % NB
        pltpu.make_async_copy(rows_v.at[b], acc_sh.at[di_v.at[j]],
                              ssems[b]).wait()
    plsc.subcore_barrier()
    nfull = ZR // CHUNK
    rem = ZR % CHUNK
    for q in range(nfull):
        pltpu.sync_copy(acc_sh.at[pl.ds(sid * ZR + q * CHUNK, CHUNK)],
                        rows_v.at[q])
        pltpu.sync_copy(rows_v.at[q],
                        out_hbm.at[pl.ds(cid * NACC + sid * ZR + q * CHUNK,
                                         CHUNK)])
    if rem:
        pltpu.sync_copy(acc_sh.at[pl.ds(sid * ZR + nfull * CHUNK, rem)],
                        rows_v.at[nfull].at[pl.ds(0, rem)])
        pltpu.sync_copy(rows_v.at[nfull].at[pl.ds(0, rem)],
                        out_hbm.at[pl.ds(cid * NACC + sid * ZR + nfull * CHUNK,
                                         rem)])


RB = 1000
NRB_G = N // RB


def _dinv_body(deg_ref, out_ref):
    deg = deg_ref[0, :] + deg_ref[1, :] + 1.0
    out_ref[...] = lax.rsqrt(deg)[:, None]


def _hp1_body(x_ref, w_ref, dinv_ref, out_ref, pk_ref):
    h = jnp.dot(x_ref[...], w_ref[...], preferred_element_type=jnp.float32)
    hp = h * dinv_ref[...]
    out_ref[...] = hp
    pk_ref[...] = hp.astype(jnp.bfloat16)


def _comb1_body(agg_ref, hp1_ref, dinv_ref, b1_ref, out_ref, pk_ref):
    s = (agg_ref[0].astype(jnp.float32) + agg_ref[1].astype(jnp.float32)
         + hp1_ref[...])
    a1 = jnp.maximum(s * dinv_ref[...] + b1_ref[...], 0.0)
    hp = a1 * dinv_ref[...]
    out_ref[...] = hp
    pk_ref[...] = hp.astype(jnp.bfloat16)


def _hfin_body(agg_ref, hp2_ref, dinv_ref, w2_ref, b2_ref, out_ref):
    s = (agg_ref[0].astype(jnp.float32) + agg_ref[1].astype(jnp.float32)
         + hp2_ref[...]) * dinv_ref[...]
    out_ref[...] = (jnp.dot(s, w2_ref[...], preferred_element_type=jnp.float32)
                    + b2_ref[...])


SRB = 400
SNRB = N // SRB
LW = 128
NTC = T * TN


def _sink_body(h_ref, ft_ref, wexp_ref, blin_ref, out_ref, k_ref, x2_ref):
    ft = ft_ref[...]
    f2 = jnp.sum(ft * ft, axis=0, keepdims=True)
    ca = lax.broadcasted_iota(jnp.int32, (LW, LW), 0)
    cb = lax.broadcasted_iota(jnp.int32, (LW, LW), 1)
    seg = jnp.where((ca // TN == cb // TN) & (ca < NTC) & (cb < NTC), 1.0, 0.0)
    cmask = lax.broadcasted_iota(jnp.int32, (1, LW), 1) < NTC

    def pass_a(r, colm):
        hb = h_ref[pl.ds(r * SRB, SRB), :]
        x2b = jnp.sum(hb * hb, axis=1, keepdims=True)
        x2_ref[pl.ds(r * SRB, SRB), :] = x2b
        g = jnp.dot(hb, ft, preferred_element_type=jnp.float32)
        mb = jnp.maximum(x2b + f2 - 2.0 * g, 0.0)
        mb = jnp.where(cmask, mb, 0.0)
        return colm + jnp.sum(mb, axis=0, keepdims=True)

    colm = lax.fori_loop(0, SNRB, pass_a, jnp.zeros((1, LW), jnp.float32))
    segm = jnp.dot(colm, seg, preferred_element_type=jnp.float32)
    mean_t = segm / jnp.float32(N * TN)
    minv = jnp.where(cmask, 1.0 / (EPS * (mean_t + 1e-8)), 0.0)

    def pass_b(r, carry):
        hb = h_ref[pl.ds(r * SRB, SRB), :]
        x2b = x2_ref[pl.ds(r * SRB, SRB), :]
        g = jnp.dot(hb, ft, preferred_element_type=jnp.float32)
        mb = jnp.maximum(x2b + f2 - 2.0 * g, 0.0)
        kb = jnp.where(cmask, jnp.exp(-mb * minv), 0.0)
        k_ref[pl.ds(r * SRB, SRB), :] = kb.astype(jnp.bfloat16)
        return carry

    lax.fori_loop(0, SNRB, pass_b, 0)

    a_marg = jnp.float32(1.0 / N)
    b_marg = jnp.float32(1.0 / TN)

    def one_iter(it, carry):
        v, _ = carry
        vseg = (seg * v.reshape(LW, 1)).astype(jnp.bfloat16)

        def blocks(r, ktu):
            kb = k_ref[pl.ds(r * SRB, SRB), :]
            kv = jnp.dot(kb, vseg, preferred_element_type=jnp.float32)
            ub = a_marg / (kv + 1e-12)
            return ktu + jnp.sum(kb * ub, axis=0, keepdims=True)

        ktu = lax.fori_loop(0, SNRB, blocks, jnp.zeros((1, LW), jnp.float32))
        v_new = jnp.where(cmask, b_marg / (ktu + 1e-12), 0.0)
        return (v_new, v)

    v0 = jnp.where(cmask, b_marg, 0.0).astype(jnp.float32)
    v_fin, v_prev = lax.fori_loop(0, SINK_ITERS, one_iter, (v0, v0))

    vseg_p = (seg * v_prev.reshape(LW, 1)).astype(jnp.bfloat16)

    def final(r, dacc):
        hb = h_ref[pl.ds(r * SRB, SRB), :]
        x2b = x2_ref[pl.ds(r * SRB, SRB), :]
        g = jnp.dot(hb, ft, preferred_element_type=jnp.float32)
        mb = jnp.maximum(x2b + f2 - 2.0 * g, 0.0)
        mb = mb * minv * EPS
        kb = k_ref[pl.ds(r * SRB, SRB), :]
        kv = jnp.dot(kb, vseg_p, preferred_element_type=jnp.float32)
        ub = a_marg / (kv + 1e-12)
        return dacc + jnp.sum(kb * ub * mb, axis=0, keepdims=True)

    dcol = lax.fori_loop(0, SNRB, final, jnp.zeros((1, LW), jnp.float32))
    dw = dcol * v_fin
    out_ref[...] = (jnp.dot(dw, wexp_ref[...], preferred_element_type=jnp.float32)
                    + blin_ref[...])


def _tc_kernels():
    f32 = jnp.float32
    dinv_fn = pl.pallas_call(
        _dinv_body,
        out_shape=jax.ShapeDtypeStruct((NACC, 1), f32),
    )
    hp1_fn = pl.pallas_call(
        _hp1_body,
        grid=(NRB_G,),
        in_specs=[
            pl.BlockSpec((RB, D), lambda i: (i, 0)),
            pl.BlockSpec((D, H), lambda i: (0, 0)),
            pl.BlockSpec((RB, 1), lambda i: (i, 0)),
        ],
        out_specs=[pl.BlockSpec((RB, H), lambda i: (i, 0)),
                   pl.BlockSpec((RB, H), lambda i: (i, 0))],
        out_shape=[jax.ShapeDtypeStruct((N, H), f32),
                   jax.ShapeDtypeStruct((N, H), jnp.bfloat16)],
    )
    comb1_fn = pl.pallas_call(
        _comb1_body,
        grid=(NRB_G,),
        in_specs=[
            pl.BlockSpec((NC, RB, H), lambda i: (0, i, 0)),
            pl.BlockSpec((RB, H), lambda i: (i, 0)),
            pl.BlockSpec((RB, 1), lambda i: (i, 0)),
            pl.BlockSpec((1, H), lambda i: (0, 0)),
        ],
        out_specs=[pl.BlockSpec((RB, H), lambda i: (i, 0)),
                   pl.BlockSpec((RB, H), lambda i: (i, 0))],
        out_shape=[jax.ShapeDtypeStruct((N, H), f32),
                   jax.ShapeDtypeStruct((N, H), jnp.bfloat16)],
    )
    hfin_fn = pl.pallas_call(
        _hfin_body,
        grid=(NRB_G,),
        in_specs=[
            pl.BlockSpec((NC, RB, H), lambda i: (0, i, 0)),
            pl.BlockSpec((RB, H), lambda i: (i, 0)),
            pl.BlockSpec((RB, 1), lambda i: (i, 0)),
            pl.BlockSpec((H, D), lambda i: (0, 0)),
            pl.BlockSpec((1, D), lambda i: (0, 0)),
        ],
        out_specs=pl.BlockSpec((RB, D), lambda i: (i, 0)),
        out_shape=jax.ShapeDtypeStruct((N, D), f32),
    )
    sink_fn = pl.pallas_call(
        _sink_body,
        out_shape=jax.ShapeDtypeStruct((1, NCLS), f32),
        scratch_shapes=[
            pltpu.VMEM((N, LW), jnp.bfloat16),
            pltpu.VMEM((N, 1), f32),
        ],
        compiler_params=pltpu.CompilerParams(vmem_limit_bytes=100 * 2**20),
    )
    return dinv_fn, hp1_fn, comb1_fn, hfin_fn, sink_fn


_DINV, _HP1, _COMB1, _HFIN, _SINK = _tc_kernels()


@functools.lru_cache(maxsize=1)
def _sc_kernels():
    mesh = plsc.VectorSubcoreMesh(core_axis_name="c", subcore_axis_name="s",
                                  num_cores=NC, num_subcores=NS)
    sc_params = pltpu.CompilerParams(use_tc_tiling_on_sc=False,
                                     needs_layout_passes=False)
    deg_fn = pl.kernel(
        _deg_body,
        out_type=jax.ShapeDtypeStruct((NC * NACC,), jnp.float32),
        mesh=mesh,
        compiler_params=sc_params,
        scratch_types=[
            pltpu.VMEM((NCHUNK, CHUNK), jnp.int32),
            pltpu.VMEM((CHUNK,), jnp.float32),
            pltpu.VMEM((ZR,), jnp.float32),
            pltpu.VMEM_SHARED((NACC,), jnp.float32),
        ],
    )
    agg_fn = pl.kernel(
        _agg_body,
        out_type=jax.ShapeDtypeStruct((NC * NACC, H), jnp.bfloat16),
        mesh=mesh,
        compiler_params=sc_params,
        scratch_types=[
            pltpu.VMEM((NCHUNK, CHUNK), jnp.int32),
            pltpu.VMEM((NCHUNK, CHUNK), jnp.int32),
            pltpu.VMEM((NB, CHUNK, H), jnp.bfloat16),
            pltpu.VMEM_SHARED((NACC, H), jnp.bfloat16),
            pltpu.VMEM_SHARED((NACC, H), jnp.bfloat16),
        ] + [pltpu.SemaphoreType.DMA] * (2 * NB),
    )
    return deg_fn, agg_fn


def kernel(x, edge_index, W1, b1, W2, b2, templates_features, W_lin, b_lin):
    _deg_kernel, _agg_kernel = _sc_kernels()
    f32 = jnp.float32
    src = edge_index[0]
    dst = edge_index[1]
    pad = E_PAD - E
    srcp = jnp.concatenate([src, jnp.zeros((pad,), jnp.int32)]).reshape(NW, NCHUNK, CHUNK)
    pad_dst = N + (jnp.arange(pad, dtype=jnp.int32) % (NACC - N))
    dstp = jnp.concatenate([dst, pad_dst]).reshape(NW, NCHUNK, CHUNK)

    degp = _deg_kernel(dstp).reshape(NC, NACC)
    dinv = _DINV(degp)

    hp1, hpb1 = _HP1(x, W1, dinv)
    agg1 = _agg_kernel(hpb1, srcp, dstp).reshape(NC, NACC, H)
    hp2, hpb2 = _COMB1(agg1, hp1, dinv, b1.reshape(1, H))
    agg2 = _agg_kernel(hpb2, srcp, dstp).reshape(NC, NACC, H)
    h = _HFIN(agg2, hp2, dinv, W2, b2.reshape(1, D))

    ftmpl = templates_features.reshape(NTC, D).T
    ft = jnp.zeros((D, LW), f32).at[:, :NTC].set(ftmpl)
    wexp = jnp.zeros((LW, NCLS), f32).at[:NTC].set(jnp.repeat(W_lin, TN, axis=0))
    out = _SINK(h, ft, wexp, b_lin.reshape(1, NCLS))
    return out.reshape(NCLS)


# speedup vs baseline: 1.0940x; 1.0940x over previous
import functools

import jax
import jax.numpy as jnp
from jax import lax
from jax.experimental import pallas as pl
from jax.experimental.pallas import tpu as pltpu
from jax.experimental.pallas import tpu_sc as plsc

N = 10000
E = 320000
D = 128
H = 64
T = 10
TN = 10
NCLS = 10
EPS = 0.1
SINK_ITERS = 20

NC = 2
NS = 16
NW = NC * NS

CHUNK = 128
NCHUNK = 80
EPW = NCHUNK * CHUNK
E_PAD = EPW * NW
NACC = N + 112
ZR = NACC // NS

def _deg_body(ei_hbm, out_hbm, idx_v, ones_v, zer_v, acc_sh):
    cid = lax.axis_index("c")
    sid = lax.axis_index("s")
    wid = cid * NS + sid
    for k in range(CHUNK // 16):
        ones_v[pl.ds(k * 16, 16)] = jnp.ones((16,), jnp.float32)

    for k in range(ZR // 16):
        zer_v[pl.ds(k * 16, 16)] = jnp.zeros((16,), jnp.float32)
    if ZR % 16:
        zer_v[pl.ds(ZR - 16, 16)] = jnp.zeros((16,), jnp.float32)

    pltpu.sync_copy(zer_v, acc_sh.at[pl.ds(sid * ZR, ZR)])
    plsc.subcore_barrier()
    pltpu.sync_copy(ei_hbm.at[1, wid], idx_v)

    def body(j, carry):
        pltpu.sync_copy(ones_v, acc_sh.at[idx_v.at[j]], add=True)
        return carry

    lax.fori_loop(0, NCHUNK, body, 0)
    plsc.subcore_barrier()
    pltpu.sync_copy(acc_sh.at[pl.ds(sid * ZR, ZR)], zer_v)
    pltpu.sync_copy(zer_v, out_hbm.at[cid, pl.ds(sid * ZR, ZR)])


NB = 8
LOOK = 4


HPW = H // 2


def _stage_rows(hpb_hbm, hpb_sh, rows_v, base, nrows):
    for q in range(nrows // CHUNK):
        pltpu.sync_copy(hpb_hbm.at[pl.ds(base + q * CHUNK, CHUNK)],
                        rows_v.at[q % NB])
        pltpu.sync_copy(rows_v.at[q % NB],
                        hpb_sh.at[pl.ds(base + q * CHUNK, CHUNK)])
    rem = nrows % CHUNK
    if rem:
        full = (nrows // CHUNK) * CHUNK
        pltpu.sync_copy(hpb_hbm.at[pl.ds(base + full, rem)],
                        rows_v.at[0].at[pl.ds(0, rem)])
        pltpu.sync_copy(rows_v.at[0].at[pl.ds(0, rem)],
                        hpb_sh.at[pl.ds(base + full, rem)])


def _agg_body(hpb_hbm, ei_hbm, out_hbm,
              si_v, di_v, rows_v, hpb_sh, acc_sh, *sems):
    gsems = sems[:NB]
    ssems = sems[NB:]
    cid = lax.axis_index("c")
    sid = lax.axis_index("s")
    wid = cid * NS + sid

    def zrow(r, carry):
        for k in range(H // 32):
            rows_v[0, r, pl.ds(k * 32, 32)] = jnp.zeros((32,), jnp.bfloat16)
        return carry

    lax.fori_loop(0, CHUNK, zrow, 0)
    for q in range(ZR // CHUNK):
        pltpu.sync_copy(rows_v.at[0],
                        acc_sh.at[pl.ds(sid * ZR + q * CHUNK, CHUNK)])
    rem = ZR % CHUNK
    if rem:
        pltpu.sync_copy(rows_v.at[0].at[pl.ds(0, rem)],
                        acc_sh.at[pl.ds(sid * ZR + (ZR // CHUNK) * CHUNK, rem)])

    @pl.when(sid == NS - 1)
    def _():
        pltpu.sync_copy(rows_v.at[0].at[pl.ds(0, NACC - N)],
                        hpb_sh.at[pl.ds(N, NACC - N)])

    @pl.when(sid < NS - 1)
    def _():
        _stage_rows(hpb_hbm, hpb_sh, rows_v, sid * ZR, ZR)

    @pl.when(sid == NS - 1)
    def _():
        _stage_rows(hpb_hbm, hpb_sh, rows_v, (NS - 1) * ZR, N - (NS - 1) * ZR)

    plsc.subcore_barrier()
    pltpu.sync_copy(ei_hbm.at[0, wid], si_v)
    pltpu.sync_copy(ei_hbm.at[1, wid], di_v)

    for j in range(LOOK):
        pltpu.async_copy(hpb_sh.at[si_v.at[j]], rows_v.at[j % NB],
                         gsems[j % NB])

    def outer(t, carry):
        for b in range(NB):
            j = t * NB + b
            pltpu.make_async_copy(hpb_sh.at[si_v.at[j]], rows_v.at[b],
                                  gsems[b]).wait()
            pltpu.async_copy(rows_v.at[b], acc_sh.at[di_v.at[j]], ssems[b],
                             add=True)
            tgt = j + LOOK
            tb = (b + LOOK) % NB

            @pl.when(tgt >= NB)
            def _():
                pltpu.make_async_copy(rows_v.at[tb],
                                      acc_sh.at[di_v.at[tgt - NB]],
                                      ssems[tb]).wait()

            @pl.when(tgt < NCHUNK)
            def _():
                pltpu.async_copy(hpb_sh.at[si_v.at[tgt]], rows_v.at[tb],
                                 gsems[tb])
        return carry

    lax.fori_loop(0, NCHUNK // NB, outer, 0)
    for j in range(NCHUNK - LOOK, NCHUNK):
        b = j % NB
        pltpu.make_async_copy(rows_v.at[b], acc_sh.at[di_v.at[j]],
                              ssems[b]).wait()
    plsc.subcore_barrier()
    nfull = ZR // CHUNK
    rem = ZR % CHUNK
    for q in range(nfull):
        pltpu.sync_copy(acc_sh.at[pl.ds(sid * ZR + q * CHUNK, CHUNK)],
                        rows_v.at[q])
        pltpu.sync_copy(rows_v.at[q],
                        out_hbm.at[cid, pl.ds(sid * ZR + q * CHUNK, CHUNK)])
    if rem:
        pltpu.sync_copy(acc_sh.at[pl.ds(sid * ZR + nfull * CHUNK, rem)],
                        rows_v.at[nfull].at[pl.ds(0, rem)])
        pltpu.sync_copy(rows_v.at[nfull].at[pl.ds(0, rem)],
                        out_hbm.at[cid, pl.ds(sid * ZR + nfull * CHUNK, rem)])


RB = 1000
NRB_G = N // RB


def _dinv_body(deg_ref, out_ref):
    deg = deg_ref[0, :] + deg_ref[1, :] + 1.0
    out_ref[...] = lax.rsqrt(deg)[:, None]


def _hp1_body(x_ref, w_ref, dinv_ref, out_ref, pk_ref):
    h = jnp.dot(x_ref[...], w_ref[...], preferred_element_type=jnp.float32)
    hp = h * dinv_ref[...]
    out_ref[...] = hp
    pk_ref[...] = hp.astype(jnp.bfloat16)


def _comb1_body(agg_ref, hp1_ref, dinv_ref, b1_ref, out_ref, pk_ref):
    s = (agg_ref[0].astype(jnp.float32) + agg_ref[1].astype(jnp.float32)
         + hp1_ref[...])
    a1 = jnp.maximum(s * dinv_ref[...] + b1_ref[...], 0.0)
    hp = a1 * dinv_ref[...]
    out_ref[...] = hp
    pk_ref[...] = hp.astype(jnp.bfloat16)


def _hfin_body(agg_ref, hp2_ref, dinv_ref, w2_ref, b2_ref, out_ref):
    s = (agg_ref[0].astype(jnp.float32) + agg_ref[1].astype(jnp.float32)
         + hp2_ref[...]) * dinv_ref[...]
    out_ref[...] = (jnp.dot(s, w2_ref[...], preferred_element_type=jnp.float32)
                    + b2_ref[...])


SRB = 400
SNRB = N // SRB
LW = 128
NTC = T * TN


def _sink_body(h_ref, ft_ref, wexp_ref, blin_ref, out_ref, k_ref, m_ref,
               x2_ref):
    ft = ft_ref[...]
    f2 = jnp.sum(ft * ft, axis=0, keepdims=True)
    ca = lax.broadcasted_iota(jnp.int32, (LW, LW), 0)
    cb = lax.broadcasted_iota(jnp.int32, (LW, LW), 1)
    seg = jnp.where((ca // TN == cb // TN) & (ca < NTC) & (cb < NTC), 1.0, 0.0)
    cmask = lax.broadcasted_iota(jnp.int32, (1, LW), 1) < NTC

    def pass_a(r, colm):
        hb = h_ref[pl.ds(r * SRB, SRB), :]
        x2b = jnp.sum(hb * hb, axis=1, keepdims=True)
        x2_ref[pl.ds(r * SRB, SRB), :] = x2b
        g = jnp.dot(hb, ft, preferred_element_type=jnp.float32)
        mb = jnp.maximum(x2b + f2 - 2.0 * g, 0.0)
        mb = jnp.where(cmask, mb, 0.0)
        return colm + jnp.sum(mb, axis=0, keepdims=True)

    colm = lax.fori_loop(0, SNRB, pass_a, jnp.zeros((1, LW), jnp.float32))
    segm = jnp.dot(colm, seg, preferred_element_type=jnp.float32)
    mean_t = segm / jnp.float32(N * TN)
    minv = jnp.where(cmask, 1.0 / (EPS * (mean_t + 1e-8)), 0.0)

    def pass_b(r, carry):
        hb = h_ref[pl.ds(r * SRB, SRB), :]
        x2b = x2_ref[pl.ds(r * SRB, SRB), :]
        g = jnp.dot(hb, ft, preferred_element_type=jnp.float32)
        mb = jnp.maximum(x2b + f2 - 2.0 * g, 0.0) * (minv * EPS)
        kb = jnp.where(cmask, jnp.exp(-mb * (1.0 / EPS)), 0.0)
        k_ref[pl.ds(r * SRB, SRB), :] = kb
        m_ref[pl.ds(r * SRB, SRB), :] = mb
        return carry

    lax.fori_loop(0, SNRB, pass_b, 0)

    a_marg = jnp.float32(1.0 / N)
    b_marg = jnp.float32(1.0 / TN)

    def one_iter(it, carry):
        v, _ = carry
        vseg = seg * v.reshape(LW, 1)

        def blocks(r, ktu):
            kb = k_ref[pl.ds(r * SRB, SRB), :]
            kv = jnp.dot(kb, vseg, preferred_element_type=jnp.float32)
            ub = a_marg / (kv + 1e-12)
            return ktu + jnp.sum(kb * ub, axis=0, keepdims=True)

        ktu = lax.fori_loop(0, SNRB, blocks, jnp.zeros((1, LW), jnp.float32))
        v_new = jnp.where(cmask, b_marg / (ktu + 1e-12), 0.0)
        return (v_new, v)

    v0 = jnp.where(cmask, b_marg, 0.0).astype(jnp.float32)
    v_fin, v_prev = lax.fori_loop(0, SINK_ITERS, one_iter, (v0, v0))

    vseg_p = seg * v_prev.reshape(LW, 1)

    def final(r, dacc):
        mb = m_ref[pl.ds(r * SRB, SRB), :]
        kb = k_ref[pl.ds(r * SRB, SRB), :]
        kv = jnp.dot(kb, vseg_p, preferred_element_type=jnp.float32)
        ub = a_marg / (kv + 1e-12)
        return dacc + jnp.sum(kb * ub * mb, axis=0, keepdims=True)

    dcol = lax.fori_loop(0, SNRB, final, jnp.zeros((1, LW), jnp.float32))
    dw = dcol * v_fin
    out_ref[...] = (jnp.dot(dw, wexp_ref[...], preferred_element_type=jnp.float32)
                    + blin_ref[...])


def _tc_kernels():
    f32 = jnp.float32
    dinv_fn = pl.pallas_call(
        _dinv_body,
        out_shape=jax.ShapeDtypeStruct((NACC, 1), f32),
    )
    hp1_fn = pl.pallas_call(
        _hp1_body,
        grid=(NRB_G,),
        in_specs=[
            pl.BlockSpec((RB, D), lambda i: (i, 0)),
            pl.BlockSpec((D, H), lambda i: (0, 0)),
            pl.BlockSpec((RB, 1), lambda i: (i, 0)),
        ],
        out_specs=[pl.BlockSpec((RB, H), lambda i: (i, 0)),
                   pl.BlockSpec((RB, H), lambda i: (i, 0))],
        out_shape=[jax.ShapeDtypeStruct((N, H), f32),
                   jax.ShapeDtypeStruct((N, H), jnp.bfloat16)],
    )
    comb1_fn = pl.pallas_call(
        _comb1_body,
        grid=(NRB_G,),
        in_specs=[
            pl.BlockSpec((NC, RB, H), lambda i: (0, i, 0)),
            pl.BlockSpec((RB, H), lambda i: (i, 0)),
            pl.BlockSpec((RB, 1), lambda i: (i, 0)),
            pl.BlockSpec((1, H), lambda i: (0, 0)),
        ],
        out_specs=[pl.BlockSpec((RB, H), lambda i: (i, 0)),
                   pl.BlockSpec((RB, H), lambda i: (i, 0))],
        out_shape=[jax.ShapeDtypeStruct((N, H), f32),
                   jax.ShapeDtypeStruct((N, H), jnp.bfloat16)],
    )
    hfin_fn = pl.pallas_call(
        _hfin_body,
        grid=(NRB_G,),
        in_specs=[
            pl.BlockSpec((NC, RB, H), lambda i: (0, i, 0)),
            pl.BlockSpec((RB, H), lambda i: (i, 0)),
            pl.BlockSpec((RB, 1), lambda i: (i, 0)),
            pl.BlockSpec((H, D), lambda i: (0, 0)),
            pl.BlockSpec((1, D), lambda i: (0, 0)),
        ],
        out_specs=pl.BlockSpec((RB, D), lambda i: (i, 0)),
        out_shape=jax.ShapeDtypeStruct((N, D), f32),
    )
    sink_fn = pl.pallas_call(
        _sink_body,
        out_shape=jax.ShapeDtypeStruct((1, NCLS), f32),
        scratch_shapes=[
            pltpu.VMEM((N, LW), f32),
            pltpu.VMEM((N, LW), f32),
            pltpu.VMEM((N, 1), f32),
        ],
        compiler_params=pltpu.CompilerParams(vmem_limit_bytes=100 * 2**20),
    )
    return dinv_fn, hp1_fn, comb1_fn, hfin_fn, sink_fn


_DINV, _HP1, _COMB1, _HFIN, _SINK = _tc_kernels()


@functools.lru_cache(maxsize=1)
def _sc_kernels():
    mesh = plsc.VectorSubcoreMesh(core_axis_name="c", subcore_axis_name="s",
                                  num_cores=NC, num_subcores=NS)
    sc_params = pltpu.CompilerParams(use_tc_tiling_on_sc=False,
                                     needs_layout_passes=False)
    deg_fn = pl.kernel(
        _deg_body,
        out_type=jax.ShapeDtypeStruct((NC, NACC), jnp.float32),
        mesh=mesh,
        compiler_params=sc_params,
        scratch_types=[
            pltpu.VMEM((NCHUNK, CHUNK), jnp.int32),
            pltpu.VMEM((CHUNK,), jnp.float32),
            pltpu.VMEM((ZR,), jnp.float32),
            pltpu.VMEM_SHARED((NACC,), jnp.float32),
        ],
    )
    agg_fn = pl.kernel(
        _agg_body,
        out_type=jax.ShapeDtypeStruct((NC, NACC, H), jnp.bfloat16),
        mesh=mesh,
        compiler_params=sc_params,
        scratch_types=[
            pltpu.VMEM((NCHUNK, CHUNK), jnp.int32),
            pltpu.VMEM((NCHUNK, CHUNK), jnp.int32),
            pltpu.VMEM((NB, CHUNK, H), jnp.bfloat16),
            pltpu.VMEM_SHARED((NACC, H), jnp.bfloat16),
            pltpu.VMEM_SHARED((NACC, H), jnp.bfloat16),
        ] + [pltpu.SemaphoreType.DMA] * (2 * NB),
    )
    return deg_fn, agg_fn


def kernel(x, edge_index, W1, b1, W2, b2, templates_features, W_lin, b_lin):
    _deg_kernel, _agg_kernel = _sc_kernels()
    f32 = jnp.float32
    epn = E // NW
    ei = jnp.pad(edge_index.reshape(2, NW, epn),
                 ((0, 0), (0, 0), (0, EPW - epn)),
                 constant_values=N).reshape(2, NW, NCHUNK, CHUNK)

    degp = _deg_kernel(ei)
    dinv = _DINV(degp)

    hp1, hpb1 = _HP1(x, W1, dinv)
    agg1 = _agg_kernel(hpb1, ei)
    hp2, hpb2 = _COMB1(agg1, hp1, dinv, b1.reshape(1, H))
    agg2 = _agg_kernel(hpb2, ei)
    h = _HFIN(agg2, hp2, dinv, W2, b2.reshape(1, D))

    ftmpl = templates_features.reshape(NTC, D).T
    ft = jnp.zeros((D, LW), f32).at[:, :NTC].set(ftmpl)
    wexp = jnp.zeros((LW, NCLS), f32).at[:NTC].set(jnp.repeat(W_lin, TN, axis=0))
    out = _SINK(h, ft, wexp, b_lin.reshape(1, NCLS))
    return out.reshape(NCLS)


# speedup vs baseline: 1.1034x; 1.0085x over previous
import functools

import jax
import jax.numpy as jnp
from jax import lax
from jax.experimental import pallas as pl
from jax.experimental.pallas import tpu as pltpu
from jax.experimental.pallas import tpu_sc as plsc

N = 10000
E = 320000
D = 128
H = 64
T = 10
TN = 10
NCLS = 10
EPS = 0.1
SINK_ITERS = 20

NC = 2
NS = 16
NW = NC * NS

CHUNK = 128
NCHUNK = 80
EPW = NCHUNK * CHUNK
E_PAD = EPW * NW
NACC = N + 112
ZR = NACC // NS

def _deg_body(ei_hbm, out_hbm, idx_v, ones_v, zer_v, acc_sh):
    cid = lax.axis_index("c")
    sid = lax.axis_index("s")
    wid = cid * NS + sid
    for k in range(CHUNK // 16):
        ones_v[pl.ds(k * 16, 16)] = jnp.ones((16,), jnp.float32)

    for k in range(ZR // 16):
        zer_v[pl.ds(k * 16, 16)] = jnp.zeros((16,), jnp.float32)
    if ZR % 16:
        zer_v[pl.ds(ZR - 16, 16)] = jnp.zeros((16,), jnp.float32)

    pltpu.sync_copy(zer_v, acc_sh.at[pl.ds(sid * ZR, ZR)])
    plsc.subcore_barrier()
    pltpu.sync_copy(ei_hbm.at[1, wid], idx_v)

    def body(j, carry):
        pltpu.sync_copy(ones_v, acc_sh.at[idx_v.at[j]], add=True)
        return carry

    lax.fori_loop(0, NCHUNK, body, 0)
    plsc.subcore_barrier()
    pltpu.sync_copy(acc_sh.at[pl.ds(sid * ZR, ZR)], zer_v)
    pltpu.sync_copy(zer_v, out_hbm.at[cid, pl.ds(sid * ZR, ZR)])


NB = 8
LOOK = 4


HPW = H // 2


def _stage_rows(hpb_hbm, hpb_sh, rows_v, base, nrows):
    for q in range(nrows // CHUNK):
        pltpu.sync_copy(hpb_hbm.at[pl.ds(base + q * CHUNK, CHUNK)],
                        rows_v.at[q % NB])
        pltpu.sync_copy(rows_v.at[q % NB],
                        hpb_sh.at[pl.ds(base + q * CHUNK, CHUNK)])
    rem = nrows % CHUNK
    if rem:
        full = (nrows // CHUNK) * CHUNK
        pltpu.sync_copy(hpb_hbm.at[pl.ds(base + full, rem)],
                        rows_v.at[0].at[pl.ds(0, rem)])
        pltpu.sync_copy(rows_v.at[0].at[pl.ds(0, rem)],
                        hpb_sh.at[pl.ds(base + full, rem)])


def _agg_body(hpb_hbm, ei_hbm, out_hbm,
              si_v, di_v, rows_v, hpb_sh, acc_sh, *sems):
    gsems = sems[:NB]
    ssems = sems[NB:]
    cid = lax.axis_index("c")
    sid = lax.axis_index("s")
    wid = cid * NS + sid

    def zrow(r, carry):
        for k in range(H // 32):
            rows_v[0, r, pl.ds(k * 32, 32)] = jnp.zeros((32,), jnp.bfloat16)
        return carry

    lax.fori_loop(0, CHUNK, zrow, 0)
    for q in range(ZR // CHUNK):
        pltpu.sync_copy(rows_v.at[0],
                        acc_sh.at[pl.ds(sid * ZR + q * CHUNK, CHUNK)])
    rem = ZR % CHUNK
    if rem:
        pltpu.sync_copy(rows_v.at[0].at[pl.ds(0, rem)],
                        acc_sh.at[pl.ds(sid * ZR + (ZR // CHUNK) * CHUNK, rem)])

    @pl.when(sid == NS - 1)
    def _():
        pltpu.sync_copy(rows_v.at[0].at[pl.ds(0, NACC - N)],
                        hpb_sh.at[pl.ds(N, NACC - N)])

    @pl.when(sid < NS - 1)
    def _():
        _stage_rows(hpb_hbm, hpb_sh, rows_v, sid * ZR, ZR)

    @pl.when(sid == NS - 1)
    def _():
        _stage_rows(hpb_hbm, hpb_sh, rows_v, (NS - 1) * ZR, N - (NS - 1) * ZR)

    plsc.subcore_barrier()
    pltpu.sync_copy(ei_hbm.at[0, wid], si_v)
    pltpu.sync_copy(ei_hbm.at[1, wid], di_v)

    for j in range(LOOK):
        pltpu.async_copy(hpb_sh.at[si_v.at[j]], rows_v.at[j % NB],
                         gsems[j % NB])

    def outer(t, carry):
        for b in range(NB):
            j = t * NB + b
            pltpu.make_async_copy(hpb_sh.at[si_v.at[j]], rows_v.at[b],
                                  gsems[b]).wait()
            pltpu.async_copy(rows_v.at[b], acc_sh.at[di_v.at[j]], ssems[b],
                             add=True)
            tgt = j + LOOK
            tb = (b + LOOK) % NB

            @pl.when(tgt >= NB)
            def _():
                pltpu.make_async_copy(rows_v.at[tb],
                                      acc_sh.at[di_v.at[tgt - NB]],
                                      ssems[tb]).wait()

            @pl.when(tgt < NCHUNK)
            def _():
                pltpu.async_copy(hpb_sh.at[si_v.at[tgt]], rows_v.at[tb],
                                 gsems[tb])
        return carry

    lax.fori_loop(0, NCHUNK // NB, outer, 0)
    for j in range(NCHUNK - LOOK, NCHUNK):
        b = j % NB
        pltpu.make_async_copy(rows_v.at[b], acc_sh.at[di_v.at[j]],
                              ssems[b]).wait()
    plsc.subcore_barrier()
    nfull = ZR // CHUNK
    rem = ZR % CHUNK
    for q in range(nfull):
        pltpu.sync_copy(acc_sh.at[pl.ds(sid * ZR + q * CHUNK, CHUNK)],
                        rows_v.at[q])
        pltpu.sync_copy(rows_v.at[q],
                        out_hbm.at[cid, pl.ds(sid * ZR + q * CHUNK, CHUNK)])
    if rem:
        pltpu.sync_copy(acc_sh.at[pl.ds(sid * ZR + nfull * CHUNK, rem)],
                        rows_v.at[nfull].at[pl.ds(0, rem)])
        pltpu.sync_copy(rows_v.at[nfull].at[pl.ds(0, rem)],
                        out_hbm.at[cid, pl.ds(sid * ZR + nfull * CHUNK, rem)])


RB = 1000
NRB_G = N // RB


def _dinv_body(deg_ref, out_ref):
    deg = deg_ref[0, :] + deg_ref[1, :] + 1.0
    out_ref[...] = lax.rsqrt(deg)[:, None]


def _hp1_body(x_ref, w_ref, dinv_ref, out_ref, pk_ref):
    h = jnp.dot(x_ref[...], w_ref[...], preferred_element_type=jnp.float32)
    hp = h * dinv_ref[...]
    out_ref[...] = hp
    pk_ref[...] = hp.astype(jnp.bfloat16)


def _comb1_body(agg_ref, hp1_ref, dinv_ref, b1_ref, out_ref, pk_ref):
    s = (agg_ref[0].astype(jnp.float32) + agg_ref[1].astype(jnp.float32)
         + hp1_ref[...])
    a1 = jnp.maximum(s * dinv_ref[...] + b1_ref[...], 0.0)
    hp = a1 * dinv_ref[...]
    out_ref[...] = hp
    pk_ref[...] = hp.astype(jnp.bfloat16)


def _hfin_body(agg_ref, hp2_ref, dinv_ref, w2_ref, b2_ref, out_ref):
    s = (agg_ref[0].astype(jnp.float32) + agg_ref[1].astype(jnp.float32)
         + hp2_ref[...]) * dinv_ref[...]
    out_ref[...] = (jnp.dot(s, w2_ref[...], preferred_element_type=jnp.float32)
                    + b2_ref[...])


SRB = 400
SNRB = N // SRB
LW = 128
NTC = T * TN


def _sink_body(h_ref, ft_ref, wexp_ref, blin_ref, out_ref, k_ref, m_ref,
               x2_ref):
    ft = ft_ref[...]
    f2 = jnp.sum(ft * ft, axis=0, keepdims=True)
    ca = lax.broadcasted_iota(jnp.int32, (LW, LW), 0)
    cb = lax.broadcasted_iota(jnp.int32, (LW, LW), 1)
    seg = jnp.where((ca // TN == cb // TN) & (ca < NTC) & (cb < NTC), 1.0, 0.0)
    cmask = lax.broadcasted_iota(jnp.int32, (1, LW), 1) < NTC

    def pass_a(r, colm):
        hb = h_ref[pl.ds(r * SRB, SRB), :]
        x2b = jnp.sum(hb * hb, axis=1, keepdims=True)
        x2_ref[pl.ds(r * SRB, SRB), :] = x2b
        g = jnp.dot(hb, ft, preferred_element_type=jnp.float32)
        mb = jnp.maximum(x2b + f2 - 2.0 * g, 0.0)
        mb = jnp.where(cmask, mb, 0.0)
        return colm + jnp.sum(mb, axis=0, keepdims=True)

    colm = lax.fori_loop(0, SNRB, pass_a, jnp.zeros((1, LW), jnp.float32))
    segm = jnp.dot(colm, seg, preferred_element_type=jnp.float32)
    mean_t = segm / jnp.float32(N * TN)
    minv = jnp.where(cmask, 1.0 / (EPS * (mean_t + 1e-8)), 0.0)

    def pass_b(r, carry):
        hb = h_ref[pl.ds(r * SRB, SRB), :]
        x2b = x2_ref[pl.ds(r * SRB, SRB), :]
        g = jnp.dot(hb, ft, preferred_element_type=jnp.float32)
        mb = jnp.maximum(x2b + f2 - 2.0 * g, 0.0) * (minv * EPS)
        kb = jnp.where(cmask, jnp.exp(-mb * (1.0 / EPS)), 0.0)
        k_ref[pl.ds(r * SRB, SRB), :] = kb
        m_ref[pl.ds(r * SRB, SRB), :] = mb
        return carry

    lax.fori_loop(0, SNRB, pass_b, 0)

    a_marg = jnp.float32(1.0 / N)
    b_marg = jnp.float32(1.0 / TN)

    def one_iter(it, carry):
        v, _ = carry
        vseg = seg * v.reshape(LW, 1)

        def blocks(r, ktu8):
            kb = k_ref[pl.ds(r * SRB, SRB), :]
            kv = jnp.dot(kb, vseg, preferred_element_type=jnp.float32)
            ub = a_marg / (kv + 1e-12)
            return ktu8 + jnp.sum((kb * ub).reshape(SRB // 8, 8, LW), axis=0)

        ktu8 = lax.fori_loop(0, SNRB, blocks,
                             jnp.zeros((8, LW), jnp.float32))
        ktu = jnp.sum(ktu8, axis=0, keepdims=True)
        v_new = jnp.where(cmask, b_marg / (ktu + 1e-12), 0.0)
        return (v_new, v)

    v0 = jnp.where(cmask, b_marg, 0.0).astype(jnp.float32)
    v_fin, v_prev = lax.fori_loop(0, SINK_ITERS, one_iter, (v0, v0))

    vseg_p = seg * v_prev.reshape(LW, 1)

    def final(r, dacc):
        mb = m_ref[pl.ds(r * SRB, SRB), :]
        kb = k_ref[pl.ds(r * SRB, SRB), :]
        kv = jnp.dot(kb, vseg_p, preferred_element_type=jnp.float32)
        ub = a_marg / (kv + 1e-12)
        return dacc + jnp.sum(kb * ub * mb, axis=0, keepdims=True)

    dcol = lax.fori_loop(0, SNRB, final, jnp.zeros((1, LW), jnp.float32))
    dw = dcol * v_fin
    out_ref[...] = (jnp.dot(dw, wexp_ref[...], preferred_element_type=jnp.float32)
                    + blin_ref[...])


def _tc_kernels():
    f32 = jnp.float32
    dinv_fn = pl.pallas_call(
        _dinv_body,
        out_shape=jax.ShapeDtypeStruct((NACC, 1), f32),
    )
    hp1_fn = pl.pallas_call(
        _hp1_body,
        grid=(NRB_G,),
        in_specs=[
            pl.BlockSpec((RB, D), lambda i: (i, 0)),
            pl.BlockSpec((D, H), lambda i: (0, 0)),
            pl.BlockSpec((RB, 1), lambda i: (i, 0)),
        ],
        out_specs=[pl.BlockSpec((RB, H), lambda i: (i, 0)),
                   pl.BlockSpec((RB, H), lambda i: (i, 0))],
        out_shape=[jax.ShapeDtypeStruct((N, H), f32),
                   jax.ShapeDtypeStruct((N, H), jnp.bfloat16)],
    )
    comb1_fn = pl.pallas_call(
        _comb1_body,
        grid=(NRB_G,),
        in_specs=[
            pl.BlockSpec((NC, RB, H), lambda i: (0, i, 0)),
            pl.BlockSpec((RB, H), lambda i: (i, 0)),
            pl.BlockSpec((RB, 1), lambda i: (i, 0)),
            pl.BlockSpec((1, H), lambda i: (0, 0)),
        ],
        out_specs=[pl.BlockSpec((RB, H), lambda i: (i, 0)),
                   pl.BlockSpec((RB, H), lambda i: (i, 0))],
        out_shape=[jax.ShapeDtypeStruct((N, H), f32),
                   jax.ShapeDtypeStruct((N, H), jnp.bfloat16)],
    )
    hfin_fn = pl.pallas_call(
        _hfin_body,
        grid=(NRB_G,),
        in_specs=[
            pl.BlockSpec((NC, RB, H), lambda i: (0, i, 0)),
            pl.BlockSpec((RB, H), lambda i: (i, 0)),
            pl.BlockSpec((RB, 1), lambda i: (i, 0)),
            pl.BlockSpec((H, D), lambda i: (0, 0)),
            pl.BlockSpec((1, D), lambda i: (0, 0)),
        ],
        out_specs=pl.BlockSpec((RB, D), lambda i: (i, 0)),
        out_shape=jax.ShapeDtypeStruct((N, D), f32),
    )
    sink_fn = pl.pallas_call(
        _sink_body,
        out_shape=jax.ShapeDtypeStruct((1, NCLS), f32),
        scratch_shapes=[
            pltpu.VMEM((N, LW), f32),
            pltpu.VMEM((N, LW), f32),
            pltpu.VMEM((N, 1), f32),
        ],
        compiler_params=pltpu.CompilerParams(vmem_limit_bytes=100 * 2**20),
    )
    return dinv_fn, hp1_fn, comb1_fn, hfin_fn, sink_fn


_DINV, _HP1, _COMB1, _HFIN, _SINK = _tc_kernels()


@functools.lru_cache(maxsize=1)
def _sc_kernels():
    mesh = plsc.VectorSubcoreMesh(core_axis_name="c", subcore_axis_name="s",
                                  num_cores=NC, num_subcores=NS)
    sc_params = pltpu.CompilerParams(use_tc_tiling_on_sc=False,
                                     needs_layout_passes=False)
    deg_fn = pl.kernel(
        _deg_body,
        out_type=jax.ShapeDtypeStruct((NC, NACC), jnp.float32),
        mesh=mesh,
        compiler_params=sc_params,
        scratch_types=[
            pltpu.VMEM((NCHUNK, CHUNK), jnp.int32),
            pltpu.VMEM((CHUNK,), jnp.float32),
            pltpu.VMEM((ZR,), jnp.float32),
            pltpu.VMEM_SHARED((NACC,), jnp.float32),
        ],
    )
    agg_fn = pl.kernel(
        _agg_body,
        out_type=jax.ShapeDtypeStruct((NC, NACC, H), jnp.bfloat16),
        mesh=mesh,
        compiler_params=sc_params,
        scratch_types=[
            pltpu.VMEM((NCHUNK, CHUNK), jnp.int32),
            pltpu.VMEM((NCHUNK, CHUNK), jnp.int32),
            pltpu.VMEM((NB, CHUNK, H), jnp.bfloat16),
            pltpu.VMEM_SHARED((NACC, H), jnp.bfloat16),
            pltpu.VMEM_SHARED((NACC, H), jnp.bfloat16),
        ] + [pltpu.SemaphoreType.DMA] * (2 * NB),
    )
    return deg_fn, agg_fn


def kernel(x, edge_index, W1, b1, W2, b2, templates_features, W_lin, b_lin):
    _deg_kernel, _agg_kernel = _sc_kernels()
    f32 = jnp.float32
    epn = E // NW
    ei = jnp.pad(edge_index.reshape(2, NW, epn),
                 ((0, 0), (0, 0), (0, EPW - epn)),
                 constant_values=N).reshape(2, NW, NCHUNK, CHUNK)

    degp = _deg_kernel(ei)
    dinv = _DINV(degp)

    hp1, hpb1 = _HP1(x, W1, dinv)
    agg1 = _agg_kernel(hpb1, ei)
    hp2, hpb2 = _COMB1(agg1, hp1, dinv, b1.reshape(1, H))
    agg2 = _agg_kernel(hpb2, ei)
    h = _HFIN(agg2, hp2, dinv, W2, b2.reshape(1, D))

    ftmpl = templates_features.reshape(NTC, D).T
    ft = jnp.zeros((D, LW), f32).at[:, :NTC].set(ftmpl)
    wexp = jnp.zeros((LW, NCLS), f32).at[:NTC].set(jnp.repeat(W_lin, TN, axis=0))
    out = _SINK(h, ft, wexp, b_lin.reshape(1, NCLS))
    return out.reshape(NCLS)
